# baseline (speedup 1.0000x reference)
import dataclasses
import functools

import jax
import jax.numpy as jnp
from jax import lax
from jax.experimental import pallas as pl
from jax.experimental.pallas import tpu as pltpu
from jax.experimental.pallas import tpu_sc as plsc

N = 10000
E = 320000
D_IN = 128
D_HID = 256
D_OUT = 128

NP = 10240
E_TOT = E + N
NS = 16
EP = 331776
EPC = EP // NS
ACCR = 10112
STRIPE = ACCR // NS

BLK = 1024
NBI = NP // BLK

ROWW1 = D_HID // 2 + 16
ROWW2 = D_OUT // 2 + 16


def _ones_col(n):
    return jnp.where(
        lax.broadcasted_iota(jnp.int32, (n, 16), 1) == 0, 1.0, 0.0
    ).astype(jnp.float32)


def _mm1(x, w, a_s, a_d):
    def body(x_ref, w_ref, as_ref, ad_ref, t_ref, ss_ref, sd_ref):
        h = jnp.dot(x_ref[...], w_ref[...], preferred_element_type=jnp.float32)
        ones = _ones_col(BLK)
        half = D_HID // 2
        t_ref[...] = jnp.concatenate(
            [h[:, :half], ones, h[:, half:], ones], axis=1)
        ss_ref[...] = jnp.sum(h * as_ref[...][None, :], axis=1)
        sd_ref[...] = jnp.sum(h * ad_ref[...][None, :], axis=1)

    return pl.pallas_call(
        body,
        grid=(NBI,),
        in_specs=[
            pl.BlockSpec((BLK, D_IN), lambda i: (i, 0)),
            pl.BlockSpec((D_IN, D_HID), lambda i: (0, 0)),
            pl.BlockSpec((D_HID,), lambda i: (0,)),
            pl.BlockSpec((D_HID,), lambda i: (0,)),
        ],
        out_specs=[
            pl.BlockSpec((BLK, 2 * ROWW1), lambda i: (i, 0)),
            pl.BlockSpec((BLK,), lambda i: (i,)),
            pl.BlockSpec((BLK,), lambda i: (i,)),
        ],
        out_shape=[
            jax.ShapeDtypeStruct((NP, 2 * ROWW1), jnp.float32),
            jax.ShapeDtypeStruct((NP,), jnp.float32),
            jax.ShapeDtypeStruct((NP,), jnp.float32),
        ],
    )(x, w, a_s, a_d)


def _ep1_mm2(acc1, b1, w2, a_s, a_d):
    def body(a0_ref, a1_ref, b_ref, w_ref, as_ref, ad_ref,
             t_ref, ss_ref, sd_ref):
        half = D_HID // 2
        num = jnp.concatenate(
            [a0_ref[:, :half], a1_ref[:, :half]], axis=1)
        den = a0_ref[:, half:half + 1]
        x2 = jnp.maximum(num / (den + 1e-16) + b_ref[...][None, :], 0.0)
        rid = (pl.program_id(0) * BLK
               + lax.broadcasted_iota(jnp.int32, (BLK, 1), 0))
        x2 = jnp.where(rid < N, x2, 0.0)
        h = jnp.dot(x2, w_ref[...], preferred_element_type=jnp.float32)
        ones = _ones_col(BLK)
        h2 = D_OUT // 2
        t_ref[...] = jnp.concatenate(
            [h[:, :h2], ones, h[:, h2:], ones], axis=1)
        ss_ref[...] = jnp.sum(h * as_ref[...][None, :], axis=1)
        sd_ref[...] = jnp.sum(h * ad_ref[...][None, :], axis=1)

    return pl.pallas_call(
        body,
        grid=(NBI,),
        in_specs=[
            pl.BlockSpec((BLK, ROWW1), lambda i: (i, 0)),
            pl.BlockSpec((BLK, ROWW1), lambda i: (NBI + i, 0)),
            pl.BlockSpec((D_HID,), lambda i: (0,)),
            pl.BlockSpec((D_HID, D_OUT), lambda i: (0, 0)),
            pl.BlockSpec((D_OUT,), lambda i: (0,)),
            pl.BlockSpec((D_OUT,), lambda i: (0,)),
        ],
        out_specs=[
            pl.BlockSpec((BLK, 2 * ROWW2), lambda i: (i, 0)),
            pl.BlockSpec((BLK,), lambda i: (i,)),
            pl.BlockSpec((BLK,), lambda i: (i,)),
        ],
        out_shape=[
            jax.ShapeDtypeStruct((NP, 2 * ROWW2), jnp.float32),
            jax.ShapeDtypeStruct((NP,), jnp.float32),
            jax.ShapeDtypeStruct((NP,), jnp.float32),
        ],
    )(acc1, acc1, b1, w2, a_s, a_d)


def _ep2(acc2, b2):
    def body(a0_ref, a1_ref, b_ref, o_ref):
        half = D_OUT // 2
        num = jnp.concatenate(
            [a0_ref[:, :half], a1_ref[:, :half]], axis=1)
        den = a0_ref[:, half:half + 1]
        o_ref[...] = num / (den + 1e-16) + b_ref[...][None, :]

    return pl.pallas_call(
        body,
        grid=(NBI,),
        in_specs=[
            pl.BlockSpec((BLK, ROWW2), lambda i: (i, 0)),
            pl.BlockSpec((BLK, ROWW2), lambda i: (NBI + i, 0)),
            pl.BlockSpec((D_OUT,), lambda i: (0,)),
        ],
        out_specs=pl.BlockSpec((BLK, D_OUT), lambda i: (i, 0)),
        out_shape=jax.ShapeDtypeStruct((NP, D_OUT), jnp.float32),
    )(acc2, acc2, b2)


def _gat_sc(table, src, dst, ssrc, sdst, roww, kk, nbuf):
    nvec = roww // 16
    ch = EPC // kk
    assert ch % nbuf == 0, "pipeline depth must divide the chunk count"
    mesh = plsc.VectorSubcoreMesh(core_axis_name="c", subcore_axis_name="s")
    cp = pltpu.CompilerParams(
        needs_layout_passes=False, use_tc_tiling_on_sc=False)

    scratch = (
        [pltpu.VMEM((NP,), jnp.float32)] * 2
        + [pltpu.VMEM((kk,), jnp.int32)] * (4 * nbuf)
        + [pltpu.VMEM((kk, roww), jnp.float32)] * nbuf
        + [pltpu.VMEM((kk,), jnp.float32)]
        + [pltpu.VMEM_SHARED((ACCR, roww), jnp.float32)]
        + [pltpu.SemaphoreType.DMA] * (3 * nbuf)
    )

    @functools.partial(
        pl.kernel,
        out_type=jax.ShapeDtypeStruct((2 * NP, roww), jnp.float32),
        mesh=mesh,
        compiler_params=cp,
        scratch_types=scratch,
    )
    def sc_kernel(t_hbm, src_hbm, dst_hbm, ssrc_hbm, sdst_hbm, out_hbm,
                  *scr):
        ssrc_v, sdst_v = scr[0], scr[1]
        src_v = scr[2:2 + nbuf]
        dst_v = scr[2 + nbuf:2 + 2 * nbuf]
        srcoff = scr[2 + 2 * nbuf:2 + 3 * nbuf]
        dsts = scr[2 + 3 * nbuf:2 + 4 * nbuf]
        rows = scr[2 + 4 * nbuf:2 + 5 * nbuf]
        e_v = scr[2 + 5 * nbuf]
        acc = scr[3 + 5 * nbuf]
        sems = scr[4 + 5 * nbuf:]
        sem_i = sems[0:nbuf]
        sem_g = sems[nbuf:2 * nbuf]
        sem_s = sems[2 * nbuf:3 * nbuf]
        rows0 = rows[0]
        sem_g0, sem_g1 = sem_g[0], sem_g[1]
        sem_s0, sem_s1 = sem_s[0], sem_s[1]
        c = lax.axis_index("c")
        s = lax.axis_index("s")
        ebase = s * EPC

        pltpu.async_copy(ssrc_hbm, ssrc_v, sem_g0)
        pltpu.async_copy(sdst_hbm, sdst_v, sem_g1)

        @pl.loop(0, kk)
        def _zero_rows(r):
            for v in range(nvec):
                rows0[r, pl.ds(v * 16, 16)] = jnp.zeros((16,), jnp.float32)

        rem = STRIPE - (STRIPE // kk) * kk
        for bq in range(STRIPE // kk):
            pltpu.async_copy(rows0, acc.at[pl.ds(s * STRIPE + bq * kk, kk)],
                             sem_s0)
        if rem:
            pltpu.async_copy(
                rows0.at[pl.ds(0, rem)],
                acc.at[pl.ds(s * STRIPE + (STRIPE // kk) * kk, rem)],
                sem_s0)
        for bq in range(STRIPE // kk):
            pltpu.make_async_copy(
                rows0, acc.at[pl.ds(s * STRIPE + bq * kk, kk)],
                sem_s0).wait()
        if rem:
            pltpu.make_async_copy(
                rows0.at[pl.ds(0, rem)],
                acc.at[pl.ds(s * STRIPE + (STRIPE // kk) * kk, rem)],
                sem_s0).wait()

        pltpu.make_async_copy(ssrc_hbm, ssrc_v, sem_g0).wait()
        pltpu.make_async_copy(sdst_hbm, sdst_v, sem_g1).wait()

        def _mx(vec_ref):
            def mbody(i, mm):
                return jnp.maximum(mm, vec_ref[pl.ds(i * 16, 16)])
            return jnp.max(lax.fori_loop(1, NP // 16, mbody,
                                         vec_ref[pl.ds(0, 16)]))

        bsum = _mx(ssrc_v) + _mx(sdst_v)
        bval = jnp.maximum(bsum, 0.2 * bsum)
        bvec = jnp.full((16,), bval, jnp.float32)

        plsc.subcore_barrier()

        def _idx_start(j, b):
            pltpu.async_copy(src_hbm.at[pl.ds(ebase + j * kk, kk)],
                             src_v[b], sem_i[b])
            pltpu.async_copy(dst_hbm.at[pl.ds(ebase + j * kk, kk)],
                             dst_v[b], sem_i[b])

        def _idx_wait(j, b):
            pltpu.make_async_copy(src_hbm.at[pl.ds(ebase + j * kk, kk)],
                                  src_v[b], sem_i[b]).wait()
            pltpu.make_async_copy(dst_hbm.at[pl.ds(ebase + j * kk, kk)],
                                  dst_v[b], sem_i[b]).wait()

        def _mk_srcoff(b):
            for q in range(kk // 16):
                si = src_v[b][pl.ds(q * 16, 16)]
                srcoff[b][pl.ds(q * 16, 16)] = si * 2 + c

        def _gather_start(b):
            pltpu.async_copy(t_hbm.at[srcoff[b]], rows[b], sem_g[b])

        def _gather_wait(b):
            pltpu.make_async_copy(t_hbm.at[srcoff[b]], rows[b],
                                  sem_g[b]).wait()

        def _scatter_start(b):
            pltpu.async_copy(rows[b], acc.at[dsts[b]], sem_s[b], add=True)

        def _scatter_wait(b):
            pltpu.make_async_copy(rows[b], acc.at[dsts[b]], sem_s[b]).wait()

        for m in range(nbuf):
            _idx_start(m, m)
        _idx_wait(0, 0)
        _mk_srcoff(0)
        _gather_start(0)

        def _body(j, b):
            nb = (b + 1) % nbuf
            for q in range(kk // 16):
                si = src_v[b][pl.ds(q * 16, 16)]
                di = dst_v[b][pl.ds(q * 16, 16)]
                dsts[b][pl.ds(q * 16, 16)] = di
                a = (plsc.load_gather(ssrc_v, [si])
                     + plsc.load_gather(sdst_v, [di]))
                a = jnp.maximum(a, 0.2 * a) - bvec
                e16 = jnp.exp(a)
                gid = lax.iota(jnp.int32, 16) + (ebase + j * kk + q * 16)
                e16 = jnp.where(gid < E_TOT, e16, 0.0)
                e_v[pl.ds(q * 16, 16)] = e16

            @pl.when(j + nbuf < ch)
            def _():
                _idx_start(j + nbuf, b)

            @pl.when(jnp.logical_and(j + 1 >= nbuf, j + 1 < ch))
            def _():
                _scatter_wait(nb)

            @pl.when(j + 1 < ch)
            def _():
                _idx_wait(j + 1, nb)
                _mk_srcoff(nb)
                _gather_start(nb)

            _gather_wait(b)

            @pl.loop(0, kk, step=4)
            def _scale(r0):
                for u in range(4):
                    r = r0 + u
                    ev = plsc.load_gather(
                        e_v, [jnp.full((16,), r, jnp.int32)])
                    for v in range(nvec - 1):
                        sl = pl.ds(v * 16, 16)
                        rows[b][r, sl] = rows[b][r, sl] * ev
                    rows[b][r, pl.ds((nvec - 1) * 16, 16)] = jnp.where(
                        lax.iota(jnp.int32, 16) == 0, ev, 0.0)

            _scatter_start(b)

        @pl.loop(0, ch, step=nbuf)
        def _edge_chunks(g):
            for p in range(nbuf):
                _body(g + p, p)

        for m in range(ch - nbuf, ch):
            _scatter_wait(m % nbuf)

        plsc.subcore_barrier()

        for bq in range(STRIPE // kk):
            off = s * STRIPE + bq * kk
            pltpu.async_copy(acc.at[pl.ds(off, kk)],
                             out_hbm.at[pl.ds(c * NP + off, kk)], sem_s1)
        if rem:
            off2 = s * STRIPE + (STRIPE // kk) * kk
            pltpu.async_copy(acc.at[pl.ds(off2, rem)],
                             out_hbm.at[pl.ds(c * NP + off2, rem)], sem_s1)
        for bq in range(STRIPE // kk):
            off = s * STRIPE + bq * kk
            pltpu.make_async_copy(acc.at[pl.ds(off, kk)],
                                  out_hbm.at[pl.ds(c * NP + off, kk)],
                                  sem_s1).wait()
        if rem:
            off2 = s * STRIPE + (STRIPE // kk) * kk
            pltpu.make_async_copy(acc.at[pl.ds(off2, rem)],
                                  out_hbm.at[pl.ds(c * NP + off2, rem)],
                                  sem_s1).wait()

    return sc_kernel(table, src, dst, ssrc, sdst)


def kernel(edge_index, node_features, W1, a_src1, a_dst1, b1,
           W2, a_src2, a_dst2, b2):
    loop = jnp.arange(N, dtype=edge_index.dtype)
    src = jnp.concatenate([edge_index[0], loop]).astype(jnp.int32)
    dst = jnp.concatenate([edge_index[1], loop]).astype(jnp.int32)
    pad = EP - E_TOT
    src = jnp.concatenate([src, jnp.zeros((pad,), jnp.int32)])
    dst = jnp.concatenate([dst, jnp.zeros((pad,), jnp.int32)])
    x = jnp.pad(node_features.astype(jnp.float32), ((0, NP - N), (0, 0)))

    t1, ss1, sd1 = _mm1(x, W1, a_src1, a_dst1)
    acc1 = _gat_sc(t1.reshape(2 * NP, ROWW1), src, dst, ss1, sd1,
                   ROWW1, 32, 3)
    t2, ss2, sd2 = _ep1_mm2(acc1, b1, W2, a_src2, a_dst2)
    acc2 = _gat_sc(t2.reshape(2 * NP, ROWW2), src, dst, ss2, sd2,
                   ROWW2, 128, 3)
    out = _ep2(acc2, b2)
    return out[:N]


# speedup vs baseline: 1.0432x; 1.0432x over previous
import dataclasses
import functools

import jax
import jax.numpy as jnp
from jax import lax
from jax.experimental import pallas as pl
from jax.experimental.pallas import tpu as pltpu
from jax.experimental.pallas import tpu_sc as plsc

N = 10000
E = 320000
D_IN = 128
D_HID = 256
D_OUT = 128

NP = 10240
E_TOT = E + N
NS = 16
EP = 331776
EPC = EP // NS
ACCR = 10112
STRIPE = ACCR // NS

BLK = 1024
NBI = NP // BLK

ROWW1 = D_HID // 2 + 16
ROWW2 = D_OUT // 2 + 16


def _ones_col(n):
    return jnp.where(
        lax.broadcasted_iota(jnp.int32, (n, 16), 1) == 0, 1.0, 0.0
    ).astype(jnp.float32)


def _mm1(x, w, a_s, a_d):
    def body(x_ref, w_ref, as_ref, ad_ref, t_ref, ss_ref, sd_ref):
        h = jnp.dot(x_ref[...], w_ref[...], preferred_element_type=jnp.float32)
        ones = _ones_col(BLK)
        half = D_HID // 2
        t_ref[...] = jnp.concatenate(
            [h[:, :half], ones, h[:, half:], ones], axis=1)
        ss_ref[...] = jnp.sum(h * as_ref[...][None, :], axis=1)
        sd_ref[...] = jnp.sum(h * ad_ref[...][None, :], axis=1)

    return pl.pallas_call(
        body,
        grid=(NBI,),
        in_specs=[
            pl.BlockSpec((BLK, D_IN), lambda i: (i, 0)),
            pl.BlockSpec((D_IN, D_HID), lambda i: (0, 0)),
            pl.BlockSpec((D_HID,), lambda i: (0,)),
            pl.BlockSpec((D_HID,), lambda i: (0,)),
        ],
        out_specs=[
            pl.BlockSpec((BLK, 2 * ROWW1), lambda i: (i, 0)),
            pl.BlockSpec((BLK,), lambda i: (i,)),
            pl.BlockSpec((BLK,), lambda i: (i,)),
        ],
        out_shape=[
            jax.ShapeDtypeStruct((NP, 2 * ROWW1), jnp.float32),
            jax.ShapeDtypeStruct((NP,), jnp.float32),
            jax.ShapeDtypeStruct((NP,), jnp.float32),
        ],
    )(x, w, a_s, a_d)


def _ep1_mm2(acc1, b1, w2, a_s, a_d):
    def body(a0_ref, a1_ref, b_ref, w_ref, as_ref, ad_ref,
             t_ref, ss_ref, sd_ref):
        half = D_HID // 2
        num = jnp.concatenate(
            [a0_ref[:, :half], a1_ref[:, :half]], axis=1)
        den = a0_ref[:, half:half + 1]
        x2 = jnp.maximum(num / (den + 1e-16) + b_ref[...][None, :], 0.0)
        rid = (pl.program_id(0) * BLK
               + lax.broadcasted_iota(jnp.int32, (BLK, 1), 0))
        x2 = jnp.where(rid < N, x2, 0.0)
        h = jnp.dot(x2, w_ref[...], preferred_element_type=jnp.float32)
        ones = _ones_col(BLK)
        h2 = D_OUT // 2
        t_ref[...] = jnp.concatenate(
            [h[:, :h2], ones, h[:, h2:], ones], axis=1)
        ss_ref[...] = jnp.sum(h * as_ref[...][None, :], axis=1)
        sd_ref[...] = jnp.sum(h * ad_ref[...][None, :], axis=1)

    return pl.pallas_call(
        body,
        grid=(NBI,),
        in_specs=[
            pl.BlockSpec((BLK, ROWW1), lambda i: (i, 0)),
            pl.BlockSpec((BLK, ROWW1), lambda i: (NBI + i, 0)),
            pl.BlockSpec((D_HID,), lambda i: (0,)),
            pl.BlockSpec((D_HID, D_OUT), lambda i: (0, 0)),
            pl.BlockSpec((D_OUT,), lambda i: (0,)),
            pl.BlockSpec((D_OUT,), lambda i: (0,)),
        ],
        out_specs=[
            pl.BlockSpec((BLK, 2 * ROWW2), lambda i: (i, 0)),
            pl.BlockSpec((BLK,), lambda i: (i,)),
            pl.BlockSpec((BLK,), lambda i: (i,)),
        ],
        out_shape=[
            jax.ShapeDtypeStruct((NP, 2 * ROWW2), jnp.float32),
            jax.ShapeDtypeStruct((NP,), jnp.float32),
            jax.ShapeDtypeStruct((NP,), jnp.float32),
        ],
    )(acc1, acc1, b1, w2, a_s, a_d)


def _ep2(acc2, b2):
    def body(a0_ref, a1_ref, b_ref, o_ref):
        half = D_OUT // 2
        num = jnp.concatenate(
            [a0_ref[:, :half], a1_ref[:, :half]], axis=1)
        den = a0_ref[:, half:half + 1]
        o_ref[...] = num / (den + 1e-16) + b_ref[...][None, :]

    return pl.pallas_call(
        body,
        grid=(NBI,),
        in_specs=[
            pl.BlockSpec((BLK, ROWW2), lambda i: (i, 0)),
            pl.BlockSpec((BLK, ROWW2), lambda i: (NBI + i, 0)),
            pl.BlockSpec((D_OUT,), lambda i: (0,)),
        ],
        out_specs=pl.BlockSpec((BLK, D_OUT), lambda i: (i, 0)),
        out_shape=jax.ShapeDtypeStruct((NP, D_OUT), jnp.float32),
    )(acc2, acc2, b2)


def _gat_sc(table, src, dst, ssrc, sdst, roww, kk, nbuf):
    nvec = roww // 16
    ch = EPC // kk
    assert ch % nbuf == 0, "pipeline depth must divide the chunk count"
    mesh = plsc.VectorSubcoreMesh(core_axis_name="c", subcore_axis_name="s")
    cp = pltpu.CompilerParams(
        needs_layout_passes=False, use_tc_tiling_on_sc=False)

    scratch = (
        [pltpu.VMEM((NP,), jnp.float32)] * 2
        + [pltpu.VMEM((kk,), jnp.int32)] * (4 * nbuf)
        + [pltpu.VMEM((kk, roww), jnp.float32)] * nbuf
        + [pltpu.VMEM((kk,), jnp.float32)]
        + [pltpu.VMEM_SHARED((ACCR, roww), jnp.float32)]
        + [pltpu.SemaphoreType.DMA] * (3 * nbuf)
    )

    @functools.partial(
        pl.kernel,
        out_type=jax.ShapeDtypeStruct((2 * NP, roww), jnp.float32),
        mesh=mesh,
        compiler_params=cp,
        scratch_types=scratch,
    )
    def sc_kernel(t_hbm, src_hbm, dst_hbm, ssrc_hbm, sdst_hbm, out_hbm,
                  *scr):
        ssrc_v, sdst_v = scr[0], scr[1]
        src_v = scr[2:2 + nbuf]
        dst_v = scr[2 + nbuf:2 + 2 * nbuf]
        srcoff = scr[2 + 2 * nbuf:2 + 3 * nbuf]
        dsts = scr[2 + 3 * nbuf:2 + 4 * nbuf]
        rows = scr[2 + 4 * nbuf:2 + 5 * nbuf]
        e_v = scr[2 + 5 * nbuf]
        acc = scr[3 + 5 * nbuf]
        sems = scr[4 + 5 * nbuf:]
        sem_i = sems[0:nbuf]
        sem_g = sems[nbuf:2 * nbuf]
        sem_s = sems[2 * nbuf:3 * nbuf]
        rows0 = rows[0]
        sem_g0, sem_g1 = sem_g[0], sem_g[1]
        sem_s0, sem_s1 = sem_s[0], sem_s[1]
        c = lax.axis_index("c")
        s = lax.axis_index("s")
        ebase = s * EPC

        pltpu.async_copy(ssrc_hbm, ssrc_v, sem_g0)
        pltpu.async_copy(sdst_hbm, sdst_v, sem_g1)

        @pl.loop(0, kk)
        def _zero_rows(r):
            for v in range(nvec):
                rows0[r, pl.ds(v * 16, 16)] = jnp.zeros((16,), jnp.float32)

        rem = STRIPE - (STRIPE // kk) * kk
        for bq in range(STRIPE // kk):
            pltpu.async_copy(rows0, acc.at[pl.ds(s * STRIPE + bq * kk, kk)],
                             sem_s0)
        if rem:
            pltpu.async_copy(
                rows0.at[pl.ds(0, rem)],
                acc.at[pl.ds(s * STRIPE + (STRIPE // kk) * kk, rem)],
                sem_s0)
        for bq in range(STRIPE // kk):
            pltpu.make_async_copy(
                rows0, acc.at[pl.ds(s * STRIPE + bq * kk, kk)],
                sem_s0).wait()
        if rem:
            pltpu.make_async_copy(
                rows0.at[pl.ds(0, rem)],
                acc.at[pl.ds(s * STRIPE + (STRIPE // kk) * kk, rem)],
                sem_s0).wait()

        pltpu.make_async_copy(ssrc_hbm, ssrc_v, sem_g0).wait()
        pltpu.make_async_copy(sdst_hbm, sdst_v, sem_g1).wait()

        def _mx(vec_ref):
            def mbody(i, mm):
                return jnp.maximum(mm, vec_ref[pl.ds(i * 16, 16)])
            return jnp.max(lax.fori_loop(1, NP // 16, mbody,
                                         vec_ref[pl.ds(0, 16)]))

        bsum = _mx(ssrc_v) + _mx(sdst_v)
        bval = jnp.maximum(bsum, 0.2 * bsum)
        bvec = jnp.full((16,), bval, jnp.float32)

        plsc.subcore_barrier()

        def _idx_start(j, b):
            pltpu.async_copy(src_hbm.at[pl.ds(ebase + j * kk, kk)],
                             src_v[b], sem_i[b])
            pltpu.async_copy(dst_hbm.at[pl.ds(ebase + j * kk, kk)],
                             dst_v[b], sem_i[b])

        def _idx_wait(j, b):
            pltpu.make_async_copy(src_hbm.at[pl.ds(ebase + j * kk, kk)],
                                  src_v[b], sem_i[b]).wait()
            pltpu.make_async_copy(dst_hbm.at[pl.ds(ebase + j * kk, kk)],
                                  dst_v[b], sem_i[b]).wait()

        def _mk_srcoff(b):
            for q in range(kk // 16):
                si = src_v[b][pl.ds(q * 16, 16)]
                srcoff[b][pl.ds(q * 16, 16)] = si * 2 + c

        def _gather_start(b):
            pltpu.async_copy(t_hbm.at[srcoff[b]], rows[b], sem_g[b])

        def _gather_wait(b):
            pltpu.make_async_copy(t_hbm.at[srcoff[b]], rows[b],
                                  sem_g[b]).wait()

        def _scatter_start(b):
            pltpu.async_copy(rows[b], acc.at[dsts[b]], sem_s[b], add=True)

        def _scatter_wait(b):
            pltpu.make_async_copy(rows[b], acc.at[dsts[b]], sem_s[b]).wait()

        for m in range(nbuf):
            _idx_start(m, m)
        _idx_wait(0, 0)
        _mk_srcoff(0)
        _gather_start(0)

        def _body(j, b):
            nb = (b + 1) % nbuf
            for q in range(kk // 16):
                si = src_v[b][pl.ds(q * 16, 16)]
                di = dst_v[b][pl.ds(q * 16, 16)]
                dsts[b][pl.ds(q * 16, 16)] = di
                a = (plsc.load_gather(ssrc_v, [si])
                     + plsc.load_gather(sdst_v, [di]))
                a = jnp.maximum(a, 0.2 * a) - bvec
                e16 = jnp.exp(a)
                gid = lax.iota(jnp.int32, 16) + (ebase + j * kk + q * 16)
                e16 = jnp.where(gid < E_TOT, e16, 0.0)
                e_v[pl.ds(q * 16, 16)] = e16

            @pl.when(j + nbuf < ch)
            def _():
                _idx_start(j + nbuf, b)

            @pl.when(jnp.logical_and(j + 1 >= nbuf, j + 1 < ch))
            def _():
                _scatter_wait(nb)

            @pl.when(j + 1 < ch)
            def _():
                _idx_wait(j + 1, nb)
                _mk_srcoff(nb)
                _gather_start(nb)

            _gather_wait(b)

            @pl.loop(0, kk, step=4)
            def _scale(r0):
                for u in range(4):
                    r = r0 + u
                    ev = plsc.load_gather(
                        e_v, [jnp.full((16,), r, jnp.int32)])
                    for v in range(nvec - 1):
                        sl = pl.ds(v * 16, 16)
                        rows[b][r, sl] = rows[b][r, sl] * ev
                    rows[b][r, pl.ds((nvec - 1) * 16, 16)] = jnp.where(
                        lax.iota(jnp.int32, 16) == 0, ev, 0.0)

            _scatter_start(b)

        @pl.loop(0, ch, step=nbuf)
        def _edge_chunks(g):
            for p in range(nbuf):
                _body(g + p, p)

        for m in range(ch - nbuf, ch):
            _scatter_wait(m % nbuf)

        plsc.subcore_barrier()

        for bq in range(STRIPE // kk):
            off = s * STRIPE + bq * kk
            pltpu.async_copy(acc.at[pl.ds(off, kk)],
                             out_hbm.at[pl.ds(c * NP + off, kk)], sem_s1)
        if rem:
            off2 = s * STRIPE + (STRIPE // kk) * kk
            pltpu.async_copy(acc.at[pl.ds(off2, rem)],
                             out_hbm.at[pl.ds(c * NP + off2, rem)], sem_s1)
        for bq in range(STRIPE // kk):
            off = s * STRIPE + bq * kk
            pltpu.make_async_copy(acc.at[pl.ds(off, kk)],
                                  out_hbm.at[pl.ds(c * NP + off, kk)],
                                  sem_s1).wait()
        if rem:
            off2 = s * STRIPE + (STRIPE // kk) * kk
            pltpu.make_async_copy(acc.at[pl.ds(off2, rem)],
                                  out_hbm.at[pl.ds(c * NP + off2, rem)],
                                  sem_s1).wait()

    return sc_kernel(table, src, dst, ssrc, sdst)


def kernel(edge_index, node_features, W1, a_src1, a_dst1, b1,
           W2, a_src2, a_dst2, b2):
    loop = jnp.arange(N, dtype=edge_index.dtype)
    src = jnp.concatenate([edge_index[0], loop]).astype(jnp.int32)
    dst = jnp.concatenate([edge_index[1], loop]).astype(jnp.int32)
    pad = EP - E_TOT
    src = jnp.concatenate([src, jnp.zeros((pad,), jnp.int32)])
    dst = jnp.concatenate([dst, jnp.zeros((pad,), jnp.int32)])
    x = jnp.pad(node_features.astype(jnp.float32), ((0, NP - N), (0, 0)))

    t1, ss1, sd1 = _mm1(x, W1, a_src1, a_dst1)
    acc1 = _gat_sc(t1.reshape(2 * NP, ROWW1), src, dst, ss1, sd1,
                   ROWW1, 64, 2)
    t2, ss2, sd2 = _ep1_mm2(acc1, b1, W2, a_src2, a_dst2)
    acc2 = _gat_sc(t2.reshape(2 * NP, ROWW2), src, dst, ss2, sd2,
                   ROWW2, 128, 3)
    out = _ep2(acc2, b2)
    return out[:N]
